# Initial kernel scaffold; baseline (speedup 1.0000x reference)
import jax
import jax.numpy as jnp
from jax.experimental import pallas as pl


def kernel(x, tables):
    raise NotImplementedError("write your pallas kernel here")


# speedup vs baseline: 11.5121x; 11.5121x over previous
import dataclasses
import functools
import math

import jax
import jax.numpy as jnp
import numpy as np
from jax import lax
from jax._src import config as _jax_config
from jax.experimental import pallas as pl
from jax.experimental.pallas import tpu as pltpu
from jax.experimental.pallas import tpu_sc as plsc

N_MAX = 2048
N_MIN = 16
L = 16
T = 2 ** 19
F = 2
B = 131072
MASK = T - 1

PI1 = np.int32(-1640531535)
PI2 = np.int32(805459861)

NC = 2
NS = 16
NW = NC * NS
PW = B // NW
C = 256
NG = C // 16
NCHUNK = PW // C
GATHER_W = 128
NGATHER = (8 * C) // GATHER_W

_b = math.exp((math.log(N_MAX) - math.log(N_MIN)) / L)
SCALES = np.array([np.float32(N_MIN * (_b ** i)) for i in range(L)],
                  dtype=np.float32)


def _encoder_body(xT_hbm, tbl_hbm, scale_hbm, base_hbm, out_hbm,
                  xs, scale_v, base_v, idx_v, feat_v, wbuf, obuf, sem):
  wid = lax.axis_index("s") * np.int32(NC) + lax.axis_index("c")
  pbase = wid * np.int32(PW)

  for d in range(3):
    pltpu.sync_copy(xT_hbm.at[pl.ds(np.int32(d * B) + pbase, PW)],
                    xs.at[pl.ds(d * PW, PW)])
  pltpu.sync_copy(scale_hbm, scale_v)
  pltpu.sync_copy(base_hbm, base_v)

  iota = lax.iota(jnp.int32, 16)

  @pl.loop(np.int32(0), np.int32(NCHUNK), step=np.int32(1))
  def _chunk(chunk):
    coff = chunk * np.int32(C)

    @pl.loop(np.int32(0), np.int32(L), step=np.int32(1))
    def _level(l):
      lo = l * np.int32(16)
      scale = scale_v[pl.ds(lo, 16)]
      base = base_v[pl.ds(lo, 16)]

      @pl.loop(np.int32(0), np.int32(NG), step=np.int32(1))
      def _grp(g):
        off = coff + g * np.int32(16)
        ux0 = xs[pl.ds(off, 16)] * scale
        ux1 = xs[pl.ds(np.int32(PW) + off, 16)] * scale
        ux2 = xs[pl.ds(np.int32(2 * PW) + off, 16)] * scale
        i0 = ux0.astype(jnp.int32)
        i1 = ux1.astype(jnp.int32)
        i2 = ux2.astype(jnp.int32)
        d0 = ux0 - i0.astype(jnp.float32)
        d1 = ux1 - i1.astype(jnp.float32)
        d2 = ux2 - i2.astype(jnp.float32)
        a0, b0 = i0, i0 + 1
        a1, b1 = i1 * PI1, (i1 + 1) * PI1
        a2, b2 = i2 * PI2, (i2 + 1) * PI2
        w00, w10 = 1.0 - d0, d0
        w01, w11 = 1.0 - d1, d1
        w02, w12 = 1.0 - d2, d2
        for corner in range(8):
          t0 = b0 if (corner & 1) else a0
          t1 = b1 if (corner & 2) else a1
          t2 = b2 if (corner & 4) else a2
          h = ((t0 ^ t1 ^ t2) & MASK) + base
          u0 = w10 if (corner & 1) else w00
          u1 = w11 if (corner & 2) else w01
          u2 = w12 if (corner & 4) else w02
          w = (u0 * u1) * u2
          cbase = np.int32(corner * C) + g * np.int32(16)
          e0 = h * np.int32(2)
          idx_v[pl.ds(cbase, 16)] = e0
          idx_v[pl.ds(np.int32(8 * C) + cbase, 16)] = e0 + np.int32(1)
          wbuf[pl.ds(cbase, 16)] = w

      copies = []
      for j in range(2 * NGATHER):
        cp = pltpu.async_copy(
            tbl_hbm.at[idx_v.at[pl.ds(j * GATHER_W, GATHER_W)]],
            feat_v.at[pl.ds(j * GATHER_W, GATHER_W)],
            sem)
        copies.append(cp)
      for cp in copies:
        cp.wait()

      col0 = lax.shift_right_logical(base, jnp.full((16,), 18, jnp.int32))
      col1 = col0 + 1

      @pl.loop(np.int32(0), np.int32(NG), step=np.int32(1))
      def _fma(g):
        acc0 = jnp.zeros((16,), jnp.float32)
        acc1 = jnp.zeros((16,), jnp.float32)
        for corner in range(8):
          rbase = np.int32(corner * C) + g * np.int32(16)
          w = wbuf[pl.ds(rbase, 16)]
          f0 = feat_v[pl.ds(rbase, 16)]
          f1 = feat_v[pl.ds(np.int32(8 * C) + rbase, 16)]
          acc0 = acc0 + w * f0
          acc1 = acc1 + w * f1
        prow = (iota + g * np.int32(16)) * np.int32(2 * L)
        plsc.store_scatter(obuf, [prow + col0], acc0)
        plsc.store_scatter(obuf, [prow + col1], acc1)

    pltpu.sync_copy(
        obuf,
        out_hbm.at[pl.ds((pbase + coff) * np.int32(2 * L), C * 2 * L)])


@jax.jit
def kernel(x, tables):
  with _jax_config.enable_x64(False):
    return _kernel_x32(x, tables)


def _kernel_x32(x, tables):
  xT = x.T.reshape(3 * B)
  tbl = tables.reshape(L * T * F)
  scales = jnp.asarray(np.repeat(SCALES, 16))
  bases = jnp.asarray(np.repeat(np.arange(L, dtype=np.int32) * T, 16))

  mesh = plsc.VectorSubcoreMesh(core_axis_name="c", subcore_axis_name="s")
  cp = pltpu.CompilerParams(needs_layout_passes=False,
                            use_tc_tiling_on_sc=False)
  enc = functools.partial(
      pl.kernel,
      compiler_params=cp,
      out_type=jax.ShapeDtypeStruct((B * 2 * L,), jnp.float32),
      mesh=mesh,
      scratch_types=[
          pltpu.VMEM((3 * PW,), jnp.float32),
          pltpu.VMEM((L * 16,), jnp.float32),
          pltpu.VMEM((L * 16,), jnp.int32),
          pltpu.VMEM((16 * C,), jnp.int32),
          pltpu.VMEM((16 * C,), jnp.float32),
          pltpu.VMEM((8 * C,), jnp.float32),
          pltpu.VMEM((C * 2 * L,), jnp.float32),
          pltpu.SemaphoreType.DMA,
      ],
  )(_encoder_body)
  return enc(xT, tbl, scales, bases).reshape(B, 2 * L)


# speedup vs baseline: 11.7485x; 1.0205x over previous
import functools
import math

import jax
import jax.numpy as jnp
import numpy as np
from jax import lax
from jax._src import config as _jax_config
from jax.experimental import pallas as pl
from jax.experimental.pallas import tpu as pltpu
from jax.experimental.pallas import tpu_sc as plsc

N_MAX = 2048
N_MIN = 16
L = 16
T = 2 ** 19
F = 2
B = 131072
MASK = T - 1

PI1 = np.int32(-1640531535)
PI2 = np.int32(805459861)

NC = 2
NS = 16
NW = NC * NS
PW = B // NW
C = 256
NG = C // 16
NCHUNK = PW // C
GATHER_W = 128
NIDX = 16 * C
NGATHER = NIDX // GATHER_W

_b = math.exp((math.log(N_MAX) - math.log(N_MIN)) / L)
SCALES = [np.float32(N_MIN * (_b ** i)) for i in range(L)]


def _encoder_body(xT_hbm, tbl_hbm, out_hbm, xs, idx_v, feat_v, wbuf, obuf, sem):
  wid = lax.axis_index("s") * np.int32(NC) + lax.axis_index("c")
  pbase = wid * np.int32(PW)

  for d in range(3):
    pltpu.sync_copy(xT_hbm.at[pl.ds(np.int32(d * B) + pbase, PW)],
                    xs.at[pl.ds(d * PW, PW)])

  iota = lax.iota(jnp.int32, 16)

  def compute_level(l, par, coff):
    scale = SCALES[l]
    ibase = par * NIDX
    wbase = par * (8 * C)

    @pl.loop(np.int32(0), np.int32(NG), step=np.int32(1))
    def _grp(g):
      off = coff + g * np.int32(16)
      ux0 = xs[pl.ds(off, 16)] * scale
      ux1 = xs[pl.ds(np.int32(PW) + off, 16)] * scale
      ux2 = xs[pl.ds(np.int32(2 * PW) + off, 16)] * scale
      i0 = ux0.astype(jnp.int32)
      i1 = ux1.astype(jnp.int32)
      i2 = ux2.astype(jnp.int32)
      d0 = ux0 - i0.astype(jnp.float32)
      d1 = ux1 - i1.astype(jnp.float32)
      d2 = ux2 - i2.astype(jnp.float32)
      a0, b0 = i0, i0 + 1
      a1, b1 = i1 * PI1, (i1 + 1) * PI1
      a2, b2 = i2 * PI2, (i2 + 1) * PI2
      w00, w10 = 1.0 - d0, d0
      w01, w11 = 1.0 - d1, d1
      w02, w12 = 1.0 - d2, d2
      for corner in range(8):
        t0 = b0 if (corner & 1) else a0
        t1 = b1 if (corner & 2) else a1
        t2 = b2 if (corner & 4) else a2
        e0 = (((t0 ^ t1 ^ t2) & MASK) + np.int32(l * T)) * np.int32(2)
        w = ((w10 if (corner & 1) else w00)
             * (w11 if (corner & 2) else w01)
             * (w12 if (corner & 4) else w02))
        cbase = g * np.int32(16)
        idx_v[pl.ds(np.int32(ibase + corner * C) + cbase, 16)] = e0
        idx_v[pl.ds(np.int32(ibase + 8 * C + corner * C) + cbase, 16)] = (
            e0 + np.int32(1))
        wbuf[pl.ds(np.int32(wbase + corner * C) + cbase, 16)] = w

  def issue_gathers(par):
    base = par * NIDX
    return [
        pltpu.async_copy(
            tbl_hbm.at[idx_v.at[pl.ds(base + j * GATHER_W, GATHER_W)]],
            feat_v.at[pl.ds(base + j * GATHER_W, GATHER_W)],
            sem)
        for j in range(NGATHER)
    ]

  def fma_level(l, par):
    fbase = par * NIDX
    wbase = par * (8 * C)

    @pl.loop(np.int32(0), np.int32(NG), step=np.int32(1))
    def _fma(g):
      gb = g * np.int32(16)
      acc0 = jnp.zeros((16,), jnp.float32)
      acc1 = jnp.zeros((16,), jnp.float32)
      for corner in range(8):
        w = wbuf[pl.ds(np.int32(wbase + corner * C) + gb, 16)]
        f0 = feat_v[pl.ds(np.int32(fbase + corner * C) + gb, 16)]
        f1 = feat_v[pl.ds(np.int32(fbase + 8 * C + corner * C) + gb, 16)]
        acc0 = acc0 + w * f0
        acc1 = acc1 + w * f1
      prow = (iota + gb) * np.int32(2 * L)
      plsc.store_scatter(obuf, [prow + np.int32(2 * l)], acc0)
      plsc.store_scatter(obuf, [prow + np.int32(2 * l + 1)], acc1)

  @pl.loop(np.int32(0), np.int32(NCHUNK), step=np.int32(1))
  def _chunk(chunk):
    coff = chunk * np.int32(C)
    prev = None
    for l in range(L):
      par = l % 2
      compute_level(l, par, coff)
      cps = issue_gathers(par)
      if prev is not None:
        pl_, pcps = prev
        for cp in pcps:
          cp.wait()
        fma_level(pl_, pl_ % 2)
      prev = (l, cps)
    pl_, pcps = prev
    for cp in pcps:
      cp.wait()
    fma_level(pl_, pl_ % 2)

    pltpu.sync_copy(
        obuf,
        out_hbm.at[pl.ds((pbase + coff) * np.int32(2 * L), C * 2 * L)])


@jax.jit
def kernel(x, tables):
  with _jax_config.enable_x64(False):
    return _kernel_x32(x, tables)


def _kernel_x32(x, tables):
  xT = x.T.reshape(3 * B)
  tbl = tables.reshape(L * T * F)

  mesh = plsc.VectorSubcoreMesh(core_axis_name="c", subcore_axis_name="s")
  cp = pltpu.CompilerParams(needs_layout_passes=False,
                            use_tc_tiling_on_sc=False)
  enc = functools.partial(
      pl.kernel,
      compiler_params=cp,
      out_type=jax.ShapeDtypeStruct((B * 2 * L,), jnp.float32),
      mesh=mesh,
      scratch_types=[
          pltpu.VMEM((3 * PW,), jnp.float32),
          pltpu.VMEM((2 * NIDX,), jnp.int32),
          pltpu.VMEM((2 * NIDX,), jnp.float32),
          pltpu.VMEM((2 * 8 * C,), jnp.float32),
          pltpu.VMEM((C * 2 * L,), jnp.float32),
          pltpu.SemaphoreType.DMA,
      ],
  )(_encoder_body)
  return enc(xT, tbl).reshape(B, 2 * L)
